# baseline (speedup 1.0000x reference)
import jax
import jax.numpy as jnp
from jax.experimental import pallas as pl
from jax.experimental.pallas import tpu as pltpu

EMBED = 768
NEXP = 8
NTOK = 2048
DFF = 3072
BD = 256
NJ = DFF // BD


def _router_body(x_ref, wr_ref, w8_ref, lb_ref):
    x = x_ref[...]
    logits = jnp.dot(x, wr_ref[...], preferred_element_type=jnp.float32)
    m = jnp.max(logits, axis=1, keepdims=True)
    el = jnp.exp(logits - m)
    p = el / jnp.sum(el, axis=1, keepdims=True)
    idx8 = jax.lax.broadcasted_iota(jnp.int32, (NTOK, NEXP), 1)
    m1 = jnp.max(p, axis=1, keepdims=True)
    i1 = jnp.min(jnp.where(p == m1, idx8, NEXP), axis=1, keepdims=True)
    sel1 = idx8 == i1
    p2 = jnp.where(sel1, -1.0, p)
    m2 = jnp.max(p2, axis=1, keepdims=True)
    i2 = jnp.min(jnp.where(p2 == m2, idx8, NEXP), axis=1, keepdims=True)
    sel2 = idx8 == i2
    s = m1 + m2 + 1e-10
    w8_ref[...] = jnp.where(sel1, m1 / s, jnp.where(sel2, m2 / s, 0.0))
    ep = jnp.mean(p, axis=0, keepdims=True)
    lb = NEXP * jnp.sum(ep * jnp.log(ep * NEXP + 1e-10))
    lb_ref[...] = jnp.reshape(lb, (1, 1))


def _moe_body(x_ref, wg_ref, wu_ref, wd_ref, w8_ref, out_ref, acc_ref,
              xw_ref, wdb_ref):
    e = pl.program_id(0)
    j = pl.program_id(1)

    @pl.when(jnp.logical_and(e == 0, j == 0))
    def _():
        out_ref[...] = jnp.zeros_like(out_ref)

    @pl.when(jnp.logical_and(e == 0, j == 1))
    def _():
        wdb_ref[...] = wd_ref[...].astype(jnp.bfloat16)

    @pl.when(j == 0)
    def _():
        ohe = (jax.lax.broadcasted_iota(jnp.int32, (1, NEXP), 1) == e
               ).astype(jnp.float32)
        wcol = jnp.sum(w8_ref[...] * ohe, axis=1, keepdims=True)
        xw_ref[...] = (x_ref[...].astype(jnp.float32) * wcol
                       ).astype(jnp.bfloat16)

    x = x_ref[...]
    g = jnp.dot(x, wg_ref[...].astype(jnp.bfloat16),
                preferred_element_type=jnp.float32)
    u = jnp.dot(xw_ref[...], wu_ref[...].astype(jnp.bfloat16),
                preferred_element_type=jnp.float32)
    h = (g * jax.lax.logistic(g) * u).astype(jnp.bfloat16)

    @pl.when(e == 0)
    def _():
        acc_ref[:, pl.ds(j * BD, BD)] = h

    @pl.when(jnp.logical_and(e != 0, e != NEXP - 1))
    def _():
        acc_ref[:, pl.ds(j * BD, BD)] += h

    @pl.when(e == NEXP - 1)
    def _():
        hfin = acc_ref[:, pl.ds(j * BD, BD)] + h
        out_ref[...] += jnp.dot(hfin, wdb_ref[pl.ds(j * BD, BD), :],
                                preferred_element_type=jnp.float32)


def _router_call(x_flat, W_router):
    return pl.pallas_call(
        _router_body,
        out_shape=(
            jax.ShapeDtypeStruct((NTOK, NEXP), jnp.float32),
            jax.ShapeDtypeStruct((1, 1), jnp.float32),
        ),
    )(x_flat, W_router)


def _moe_call(x_bf, W_gate, W_up, W_down, w8):
    return pl.pallas_call(
        _moe_body,
        grid=(NEXP, NJ),
        in_specs=[
            pl.BlockSpec((NTOK, EMBED), lambda e, j: (0, 0)),
            pl.BlockSpec((EMBED, BD), lambda e, j: (0, e * NJ + j)),
            pl.BlockSpec((EMBED, BD), lambda e, j: (0, e * NJ + j)),
            pl.BlockSpec((DFF, EMBED), lambda e, j: (0, 0)),
            pl.BlockSpec((NTOK, NEXP), lambda e, j: (0, 0)),
        ],
        out_specs=pl.BlockSpec((NTOK, EMBED), lambda e, j: (0, 0)),
        out_shape=jax.ShapeDtypeStruct((NTOK, EMBED), jnp.float32),
        scratch_shapes=[
            pltpu.VMEM((NTOK, DFF), jnp.bfloat16),
            pltpu.VMEM((NTOK, EMBED), jnp.bfloat16),
            pltpu.VMEM((DFF, EMBED), jnp.bfloat16),
        ],
    )(x_bf, W_gate, W_up, W_down, w8)


def kernel(x, W_router, W_gate, W_up, W_down):
    x_flat = x.reshape(NTOK, EMBED)
    w8, lb = _router_call(x_flat, W_router)
    x_bf = x_flat.astype(jnp.bfloat16)
    out = _moe_call(x_bf, W_gate, W_up, W_down, w8)
    return out.reshape(x.shape), lb[0, 0]


# speedup vs baseline: 1.0296x; 1.0296x over previous
import jax
import jax.numpy as jnp
from jax.experimental import pallas as pl
from jax.experimental.pallas import tpu as pltpu

EMBED = 768
NEXP = 8
NTOK = 2048
DFF = 3072
BD = 512
NJ = DFF // BD
NSTEP = NEXP * NJ


def _router_body(x_ref, wr_ref, w8_ref, lb_ref):
    x = x_ref[...]
    logits = jnp.dot(x, wr_ref[...], preferred_element_type=jnp.float32)
    m = jnp.max(logits, axis=1, keepdims=True)
    el = jnp.exp(logits - m)
    p = el / jnp.sum(el, axis=1, keepdims=True)
    idx8 = jax.lax.broadcasted_iota(jnp.int32, (NTOK, NEXP), 1)
    m1 = jnp.max(p, axis=1, keepdims=True)
    i1 = jnp.min(jnp.where(p == m1, idx8, NEXP), axis=1, keepdims=True)
    sel1 = idx8 == i1
    p2 = jnp.where(sel1, -1.0, p)
    m2 = jnp.max(p2, axis=1, keepdims=True)
    i2 = jnp.min(jnp.where(p2 == m2, idx8, NEXP), axis=1, keepdims=True)
    sel2 = idx8 == i2
    s = m1 + m2 + 1e-10
    w8_ref[...] = jnp.where(sel1, m1 / s, jnp.where(sel2, m2 / s, 0.0))
    ep = jnp.mean(p, axis=0, keepdims=True)
    lb = NEXP * jnp.sum(ep * jnp.log(ep * NEXP + 1e-10))
    lb_ref[...] = jnp.reshape(lb, (1, 1))


def _moe_body(xbf_ref, w8_ref, wg_hbm, wu_hbm, wd_hbm, out_ref,
              wbuf, acc_ref, xw_ref, wdf_ref,
              sems, wdsem):
    def wcopies(k, s):
        col = pl.ds(k * BD, BD)
        cg = pltpu.make_async_copy(wg_hbm.at[:, col], wbuf.at[s, 0],
                                   sems.at[s])
        cu = pltpu.make_async_copy(wu_hbm.at[:, col], wbuf.at[s, 1],
                                   sems.at[s])
        return cg, cu

    cg0, cu0 = wcopies(0, 0)
    cg0.start()
    cu0.start()
    wdcp = pltpu.make_async_copy(wd_hbm, wdf_ref, wdsem)
    wdcp.start()

    def step(k, carry):
        s = k % 2
        e = k // NJ
        j = k - e * NJ

        @pl.when(k + 1 < NSTEP)
        def _():
            cg, cu = wcopies(k + 1, (k + 1) % 2)
            cg.start()
            cu.start()

        cg, cu = wcopies(k, s)
        cg.wait()
        cu.wait()

        @pl.when(j == 0)
        def _():
            ohe = (jax.lax.broadcasted_iota(jnp.int32, (1, NEXP), 1) == e
                   ).astype(jnp.float32)
            wcol = jnp.sum(w8_ref[...] * ohe, axis=1, keepdims=True)
            xw_ref[...] = (xbf_ref[...].astype(jnp.float32) * wcol
                           ).astype(jnp.bfloat16)

        g = jnp.dot(xbf_ref[...], wbuf[s, 0].astype(jnp.bfloat16),
                    preferred_element_type=jnp.float32)
        u = jnp.dot(xw_ref[...], wbuf[s, 1].astype(jnp.bfloat16),
                    preferred_element_type=jnp.float32)
        h = (g * jax.lax.logistic(g) * u).astype(jnp.bfloat16)
        dcol = pl.ds(j * BD, BD)

        @pl.when(e == 0)
        def _():
            acc_ref[:, dcol] = h

        @pl.when(jnp.logical_and(e != 0, e != NEXP - 1))
        def _():
            acc_ref[:, dcol] += h

        @pl.when(e == NEXP - 1)
        def _():
            @pl.when(j == 0)
            def _():
                wdcp2 = pltpu.make_async_copy(wd_hbm, wdf_ref, wdsem)
                wdcp2.wait()

            hfin = acc_ref[:, dcol] + h
            y = jnp.dot(hfin, wdf_ref[dcol, :].astype(jnp.bfloat16),
                        preferred_element_type=jnp.float32)

            @pl.when(j == 0)
            def _():
                out_ref[...] = y

            @pl.when(j != 0)
            def _():
                out_ref[...] += y

        return carry

    jax.lax.fori_loop(0, NSTEP, step, 0)


def _router_call(x_flat, W_router):
    return pl.pallas_call(
        _router_body,
        out_shape=(
            jax.ShapeDtypeStruct((NTOK, NEXP), jnp.float32),
            jax.ShapeDtypeStruct((1, 1), jnp.float32),
        ),
    )(x_flat, W_router)


def _moe_call(x_bf, w8, W_gate, W_up, W_down):
    return pl.pallas_call(
        _moe_body,
        in_specs=[
            pl.BlockSpec(memory_space=pltpu.VMEM),
            pl.BlockSpec(memory_space=pltpu.VMEM),
            pl.BlockSpec(memory_space=pl.ANY),
            pl.BlockSpec(memory_space=pl.ANY),
            pl.BlockSpec(memory_space=pl.ANY),
        ],
        out_specs=pl.BlockSpec(memory_space=pltpu.VMEM),
        out_shape=jax.ShapeDtypeStruct((NTOK, EMBED), jnp.float32),
        scratch_shapes=[
            pltpu.VMEM((2, 2, EMBED, BD), jnp.float32),
            pltpu.VMEM((NTOK, DFF), jnp.bfloat16),
            pltpu.VMEM((NTOK, EMBED), jnp.bfloat16),
            pltpu.VMEM((DFF, EMBED), jnp.float32),
            pltpu.SemaphoreType.DMA((2,)),
            pltpu.SemaphoreType.DMA,
        ],
    )(x_bf, w8, W_gate, W_up, W_down)


def kernel(x, W_router, W_gate, W_up, W_down):
    x_flat = x.reshape(NTOK, EMBED)
    w8, lb = _router_call(x_flat, W_router)
    out = _moe_call(x_flat.astype(jnp.bfloat16), w8, W_gate, W_up, W_down)
    return out.reshape(x.shape), lb[0, 0]


# speedup vs baseline: 1.0508x; 1.0206x over previous
import jax
import jax.numpy as jnp
from jax.experimental import pallas as pl
from jax.experimental.pallas import tpu as pltpu

EMBED = 768
NEXP = 8
NTOK = 2048
DFF = 3072
BD = 1024
NJ = DFF // BD
RSPLIT = 4
RB = NTOK // RSPLIT
NSTEP = NEXP * NJ


def _router_body(x_ref, wr_ref, w8_ref, lb_ref):
    x = x_ref[...]
    logits = jnp.dot(x, wr_ref[...], preferred_element_type=jnp.float32)
    m = jnp.max(logits, axis=1, keepdims=True)
    el = jnp.exp(logits - m)
    p = el / jnp.sum(el, axis=1, keepdims=True)
    idx8 = jax.lax.broadcasted_iota(jnp.int32, (NTOK, NEXP), 1)
    m1 = jnp.max(p, axis=1, keepdims=True)
    i1 = jnp.min(jnp.where(p == m1, idx8, NEXP), axis=1, keepdims=True)
    sel1 = idx8 == i1
    p2 = jnp.where(sel1, -1.0, p)
    m2 = jnp.max(p2, axis=1, keepdims=True)
    i2 = jnp.min(jnp.where(p2 == m2, idx8, NEXP), axis=1, keepdims=True)
    sel2 = idx8 == i2
    s = m1 + m2 + 1e-10
    w8_ref[...] = jnp.where(sel1, m1 / s, jnp.where(sel2, m2 / s, 0.0))
    ep = jnp.mean(p, axis=0, keepdims=True)
    lb = NEXP * jnp.sum(ep * jnp.log(ep * NEXP + 1e-10))
    lb_ref[...] = jnp.reshape(lb, (1, 1))


def _moe_body(xbf_ref, w8_ref, wg_hbm, wu_hbm, wd_hbm, out_ref,
              wbuf, acc_ref, xw_ref, wdf_ref,
              sems, wdsem):
    def wcopies(k, s):
        col = pl.ds(k * BD, BD)
        cg = pltpu.make_async_copy(wg_hbm.at[:, col], wbuf.at[s, 0],
                                   sems.at[s])
        cu = pltpu.make_async_copy(wu_hbm.at[:, col], wbuf.at[s, 1],
                                   sems.at[s])
        return cg, cu

    cg0, cu0 = wcopies(0, 0)
    cg0.start()
    cu0.start()
    wdcp = pltpu.make_async_copy(wd_hbm, wdf_ref, wdsem)
    wdcp.start()

    def step(k, carry):
        s = k % 2
        e = k // NJ
        j = k - e * NJ

        @pl.when(k + 1 < NSTEP)
        def _():
            cg, cu = wcopies(k + 1, (k + 1) % 2)
            cg.start()
            cu.start()

        cg, cu = wcopies(k, s)
        cg.wait()
        cu.wait()

        @pl.when(j == 0)
        def _():
            ohe = (jax.lax.broadcasted_iota(jnp.int32, (1, NEXP), 1) == e
                   ).astype(jnp.float32)
            wcol = jnp.sum(w8_ref[...] * ohe, axis=1, keepdims=True)
            xw_ref[...] = (xbf_ref[...].astype(jnp.float32) * wcol
                           ).astype(jnp.bfloat16)

        @pl.when(jnp.logical_and(e == NEXP - 1, j == 0))
        def _():
            wdcp2 = pltpu.make_async_copy(wd_hbm, wdf_ref, wdsem)
            wdcp2.wait()

        dcol = pl.ds(j * BD, BD)
        wgb = wbuf[s, 0].astype(jnp.bfloat16)
        wub = wbuf[s, 1].astype(jnp.bfloat16)
        for r in range(RSPLIT):
            rows = pl.ds(r * RB, RB)
            g = jnp.dot(xbf_ref[rows, :], wgb,
                        preferred_element_type=jnp.float32)
            u = jnp.dot(xw_ref[rows, :], wub,
                        preferred_element_type=jnp.float32)
            h = (g * jax.lax.logistic(g) * u).astype(jnp.bfloat16)

            @pl.when(e == 0)
            def _():
                acc_ref[rows, dcol] = h

            @pl.when(jnp.logical_and(e != 0, e != NEXP - 1))
            def _():
                acc_ref[rows, dcol] += h

            @pl.when(e == NEXP - 1)
            def _():
                hfin = acc_ref[rows, dcol] + h
                y = jnp.dot(hfin, wdf_ref[dcol, :].astype(jnp.bfloat16),
                            preferred_element_type=jnp.float32)

                @pl.when(j == 0)
                def _():
                    out_ref[rows, :] = y

                @pl.when(j != 0)
                def _():
                    out_ref[rows, :] += y

        return carry

    jax.lax.fori_loop(0, NSTEP, step, 0)


def _router_call(x_flat, W_router):
    return pl.pallas_call(
        _router_body,
        out_shape=(
            jax.ShapeDtypeStruct((NTOK, NEXP), jnp.float32),
            jax.ShapeDtypeStruct((1, 1), jnp.float32),
        ),
    )(x_flat, W_router)


def _moe_call(x_bf, w8, W_gate, W_up, W_down):
    return pl.pallas_call(
        _moe_body,
        in_specs=[
            pl.BlockSpec(memory_space=pltpu.VMEM),
            pl.BlockSpec(memory_space=pltpu.VMEM),
            pl.BlockSpec(memory_space=pl.ANY),
            pl.BlockSpec(memory_space=pl.ANY),
            pl.BlockSpec(memory_space=pl.ANY),
        ],
        out_specs=pl.BlockSpec(memory_space=pltpu.VMEM),
        out_shape=jax.ShapeDtypeStruct((NTOK, EMBED), jnp.float32),
        scratch_shapes=[
            pltpu.VMEM((2, 2, EMBED, BD), jnp.float32),
            pltpu.VMEM((NTOK, DFF), jnp.bfloat16),
            pltpu.VMEM((NTOK, EMBED), jnp.bfloat16),
            pltpu.VMEM((DFF, EMBED), jnp.float32),
            pltpu.SemaphoreType.DMA((2,)),
            pltpu.SemaphoreType.DMA,
        ],
    )(x_bf, w8, W_gate, W_up, W_down)


def kernel(x, W_router, W_gate, W_up, W_down):
    x_flat = x.reshape(NTOK, EMBED)
    w8, lb = _router_call(x_flat, W_router)
    out = _moe_call(x_flat.astype(jnp.bfloat16), w8, W_gate, W_up, W_down)
    return out.reshape(x.shape), lb[0, 0]
